# baseline (speedup 1.0000x reference)
import functools

import jax
import jax.numpy as jnp
from jax import lax
from jax.experimental import pallas as pl
from jax.experimental.pallas import tpu as pltpu
from jax.experimental.pallas import tpu_sc as plsc

_NC = 2
_NS = 16
_NW = _NC * _NS

_TC = 256
_IDX_MINOR = 128


def _sc_embed_lora(x1d, w128, a_t, bt1d, n_tokens, embed_dim, rank):
  tokens_per_worker = n_tokens // _NW
  chunks = tokens_per_worker // _TC
  groups = _TC // _IDX_MINOR
  jblocks = embed_dim // 16
  tgroups = _TC // 16

  mesh = plsc.VectorSubcoreMesh(
      core_axis_name="c", subcore_axis_name="s",
      num_cores=_NC, num_subcores=_NS)

  @functools.partial(
      pl.kernel,
      out_type=jax.ShapeDtypeStruct((n_tokens, embed_dim), jnp.float32),
      mesh=mesh,
      compiler_params=pltpu.CompilerParams(use_tc_tiling_on_sc=False),
      scratch_types=[
          pltpu.VMEM((2, _TC), jnp.int32),
          pltpu.VMEM((2, _TC), jnp.int32),
          pltpu.VMEM((2, _TC, 2 * embed_dim), jnp.float32),
          pltpu.VMEM((2, _TC, rank), jnp.float32),
          pltpu.VMEM((2, _TC, embed_dim), jnp.float32),
          pltpu.VMEM((rank * embed_dim,), jnp.float32),
          pltpu.SemaphoreType.DMA((2,)),
          pltpu.SemaphoreType.DMA((2,)),
      ],
  )
  def k(x_hbm, w_hbm, at_hbm, bt_hbm, out_hbm,
        idx_v, idxw_v, base_v, a_v, out_v, bt_v, gsem, osem):
    wid = lax.axis_index("s") * _NC + lax.axis_index("c")
    pltpu.sync_copy(bt_hbm, bt_v)
    worker_tok0 = wid * tokens_per_worker

    def gather_parts(c, b):
      tok0 = pl.multiple_of(worker_tok0 + c * _TC, _TC)
      parts = []
      for g in range(groups):
        sl = pl.ds(g * _IDX_MINOR, _IDX_MINOR)
        parts.append((w_hbm.at[idxw_v.at[b].at[sl]],
                      base_v.at[b].at[sl], gsem.at[b]))
        parts.append((at_hbm.at[idx_v.at[b].at[sl]],
                      a_v.at[b].at[sl], gsem.at[b]))
      return tok0, parts

    def fire(c, b):
      tok0 = pl.multiple_of(worker_tok0 + c * _TC, _TC)
      pltpu.sync_copy(x_hbm.at[pl.ds(tok0, _TC)], idx_v.at[b])
      for i in range(_TC // 16):
        v = idx_v[b, pl.ds(i * 16, 16)]
        idxw_v[b, pl.ds(i * 16, 16)] = v >> 1
      _, parts = gather_parts(c, b)
      for src, dst, sem in parts:
        pltpu.async_copy(src, dst, sem)

    def wait_gathers(c, b):
      _, parts = gather_parts(c, b)
      for src, dst, sem in parts:
        pltpu.make_async_copy(src, dst, sem).wait()

    btv = [[bt_v[pl.ds(r * embed_dim + j * 16, 16)] for j in range(jblocks)]
           for r in range(rank)]

    def compute(b):
      def grp1(g, carry):
        parg = (idx_v[b, pl.ds(g * 16, 16)] & 1) * embed_dim
        for kk in range(16):
          t = g * 16 + kk
          paroff = parg[kk]
          av = a_v[b, t, :]
          acc = [base_v[b, t, pl.ds(paroff + j * 16, 16)]
                 for j in range(jblocks)]
          for r in range(rank // 2):
            a = av[r]
            for j in range(jblocks):
              acc[j] = acc[j] + a * btv[r][j]
          for j in range(jblocks):
            out_v[b, t, pl.ds(j * 16, 16)] = acc[j]
        return carry

      lax.fori_loop(0, tgroups, grp1, 0, unroll=False)

      def grp2(g, carry):
        for kk in range(16):
          t = g * 16 + kk
          av = a_v[b, t, :]
          acc = [out_v[b, t, pl.ds(j * 16, 16)] for j in range(jblocks)]
          for r in range(rank // 2, rank):
            a = av[r]
            for j in range(jblocks):
              acc[j] = acc[j] + a * btv[r][j]
          for j in range(jblocks):
            out_v[b, t, pl.ds(j * 16, 16)] = acc[j]
        return carry

      lax.fori_loop(0, tgroups, grp2, 0, unroll=False)

    def out_slice(c):
      tok0 = pl.multiple_of(worker_tok0 + c * _TC, _TC)
      return out_hbm.at[pl.ds(tok0, _TC)]

    fire(0, 0)

    def chunk_body(c, carry):
      b = lax.rem(c, 2)
      nb = 1 - b
      nc = jnp.minimum(c + 1, chunks - 1)
      fire(nc, nb)
      wait_gathers(c, b)

      @pl.when(c >= 2)
      def _():
        pltpu.make_async_copy(out_v.at[b], out_slice(c - 2), osem.at[b]).wait()

      compute(b)
      pltpu.async_copy(out_v.at[b], out_slice(c), osem.at[b])
      return carry

    lax.fori_loop(0, chunks, chunk_body, 0, unroll=False)

    wait_gathers(chunks - 1, chunks % 2)
    pltpu.make_async_copy(
        out_v.at[(chunks - 2) % 2], out_slice(chunks - 2),
        osem.at[(chunks - 2) % 2]).wait()
    pltpu.make_async_copy(
        out_v.at[(chunks - 1) % 2], out_slice(chunks - 1),
        osem.at[(chunks - 1) % 2]).wait()

  return k(x1d, w128, a_t, bt1d)


def _pack_rows_tc(w, blk=1024):
  v, d = w.shape

  def body(w_ref, out_ref):
    w3 = w_ref[...].reshape(blk // 2, 2, d)
    out_ref[...] = jnp.concatenate([w3[:, 0, :], w3[:, 1, :]], axis=-1)

  return pl.pallas_call(
      body,
      grid=(v // blk,),
      in_specs=[pl.BlockSpec((blk, d), lambda i: (i, 0))],
      out_specs=pl.BlockSpec((blk // 2, 2 * d), lambda i: (i, 0)),
      out_shape=jax.ShapeDtypeStruct((v // 2, 2 * d), jnp.float32),
  )(w)


def kernel(x, W_base, A, B):
  batch, seq = x.shape
  vocab, embed_dim = W_base.shape
  rank = A.shape[0]
  n_tokens = batch * seq

  x1d = x.reshape(n_tokens).astype(jnp.int32)
  w128 = _pack_rows_tc(W_base)
  a_t = A.T
  bt1d = B.T.reshape(rank * embed_dim)

  out = _sc_embed_lora(x1d, w128, a_t, bt1d, n_tokens, embed_dim, rank)
  return out.reshape(batch, seq, embed_dim)


# speedup vs baseline: 1.2818x; 1.2818x over previous
import functools

import jax
import jax.numpy as jnp
from jax import lax
from jax.experimental import pallas as pl
from jax.experimental.pallas import tpu as pltpu
from jax.experimental.pallas import tpu_sc as plsc

_NC = 2
_NS = 16
_NW = _NC * _NS

_TC = 256
_IDX_MINOR = 128


def _sc_embed_lora(x1d, w128, a_t, bt1d, n_tokens, embed_dim, rank, half_vocab):
  tokens_per_worker = n_tokens // _NW
  chunks = tokens_per_worker // _TC
  groups = _TC // _IDX_MINOR
  jblocks = embed_dim // 16
  tgroups = _TC // 16

  mesh = plsc.VectorSubcoreMesh(
      core_axis_name="c", subcore_axis_name="s",
      num_cores=_NC, num_subcores=_NS)

  @functools.partial(
      pl.kernel,
      out_type=jax.ShapeDtypeStruct((n_tokens, embed_dim), jnp.float32),
      mesh=mesh,
      compiler_params=pltpu.CompilerParams(use_tc_tiling_on_sc=False),
      scratch_types=[
          pltpu.VMEM((2, _TC), jnp.int32),
          pltpu.VMEM((2, _TC), jnp.int32),
          pltpu.VMEM((2, _TC, 2 * embed_dim), jnp.float32),
          pltpu.VMEM((2, _TC, rank), jnp.float32),
          pltpu.VMEM((2, _TC, embed_dim), jnp.float32),
          pltpu.VMEM((rank * embed_dim,), jnp.float32),
          pltpu.SemaphoreType.DMA((2,)),
          pltpu.SemaphoreType.DMA((2,)),
      ],
  )
  def k(x_hbm, w_hbm, at_hbm, bt_hbm, out_hbm,
        idx_v, idxw_v, base_v, a_v, out_v, bt_v, gsem, osem):
    wid = lax.axis_index("s") * _NC + lax.axis_index("c")
    pltpu.sync_copy(bt_hbm, bt_v)
    worker_tok0 = wid * tokens_per_worker

    def gather_parts(c, b):
      tok0 = pl.multiple_of(worker_tok0 + c * _TC, _TC)
      parts = []
      for g in range(groups):
        sl = pl.ds(g * _IDX_MINOR, _IDX_MINOR)
        parts.append((w_hbm.at[idxw_v.at[b].at[sl]],
                      base_v.at[b].at[sl], gsem.at[b]))
        parts.append((at_hbm.at[idx_v.at[b].at[sl]],
                      a_v.at[b].at[sl], gsem.at[b]))
      return tok0, parts

    def fire(c, b):
      tok0 = pl.multiple_of(worker_tok0 + c * _TC, _TC)
      pltpu.sync_copy(x_hbm.at[pl.ds(tok0, _TC)], idx_v.at[b])
      for i in range(_TC // 16):
        v = idx_v[b, pl.ds(i * 16, 16)]
        hi = 1 + ((v - half_vocab) >> 31)
        idxw_v[b, pl.ds(i * 16, 16)] = v - hi * half_vocab
      _, parts = gather_parts(c, b)
      for src, dst, sem in parts:
        pltpu.async_copy(src, dst, sem)

    def wait_gathers(c, b):
      _, parts = gather_parts(c, b)
      for src, dst, sem in parts:
        pltpu.make_async_copy(src, dst, sem).wait()

    btv = [[bt_v[pl.ds(r * embed_dim + j * 16, 16)] for j in range(jblocks)]
           for r in range(rank)]

    def compute(b):
      def grp1(g, carry):
        vg = idx_v[b, pl.ds(g * 16, 16)]
        parg = (1 + ((vg - half_vocab) >> 31)) * embed_dim
        for kk in range(16):
          t = g * 16 + kk
          paroff = parg[kk]
          av = a_v[b, t, :]
          acc = [base_v[b, t, pl.ds(paroff + j * 16, 16)]
                 for j in range(jblocks)]
          for r in range(rank // 2):
            a = av[r]
            for j in range(jblocks):
              acc[j] = acc[j] + a * btv[r][j]
          for j in range(jblocks):
            out_v[b, t, pl.ds(j * 16, 16)] = acc[j]
        return carry

      lax.fori_loop(0, tgroups, grp1, 0, unroll=False)

      def grp2(g, carry):
        for kk in range(16):
          t = g * 16 + kk
          av = a_v[b, t, :]
          acc = [out_v[b, t, pl.ds(j * 16, 16)] for j in range(jblocks)]
          for r in range(rank // 2, rank):
            a = av[r]
            for j in range(jblocks):
              acc[j] = acc[j] + a * btv[r][j]
          for j in range(jblocks):
            out_v[b, t, pl.ds(j * 16, 16)] = acc[j]
        return carry

      lax.fori_loop(0, tgroups, grp2, 0, unroll=False)

    def out_slice(c):
      tok0 = pl.multiple_of(worker_tok0 + c * _TC, _TC)
      return out_hbm.at[pl.ds(tok0, _TC)]

    fire(0, 0)

    def chunk_body(c, carry):
      b = lax.rem(c, 2)
      nb = 1 - b
      nc = jnp.minimum(c + 1, chunks - 1)
      fire(nc, nb)
      wait_gathers(c, b)

      @pl.when(c >= 2)
      def _():
        pltpu.make_async_copy(out_v.at[b], out_slice(c - 2), osem.at[b]).wait()

      compute(b)
      pltpu.async_copy(out_v.at[b], out_slice(c), osem.at[b])
      return carry

    lax.fori_loop(0, chunks, chunk_body, 0, unroll=False)

    wait_gathers(chunks - 1, chunks % 2)
    pltpu.make_async_copy(
        out_v.at[(chunks - 2) % 2], out_slice(chunks - 2),
        osem.at[(chunks - 2) % 2]).wait()
    pltpu.make_async_copy(
        out_v.at[(chunks - 1) % 2], out_slice(chunks - 1),
        osem.at[(chunks - 1) % 2]).wait()

  return k(x1d, w128, a_t, bt1d)


def _pack_rows_tc(w, blk=2000):
  v, d = w.shape
  h = v // 2

  def body(top_ref, bot_ref, out_ref):
    out_ref[...] = jnp.concatenate([top_ref[...], bot_ref[...]], axis=-1)

  return pl.pallas_call(
      body,
      grid=(h // blk,),
      in_specs=[pl.BlockSpec((blk, d), lambda i: (i, 0)),
                pl.BlockSpec((blk, d), lambda i: (i + h // blk, 0))],
      out_specs=pl.BlockSpec((blk, 2 * d), lambda i: (i, 0)),
      out_shape=jax.ShapeDtypeStruct((h, 2 * d), jnp.float32),
  )(w, w)


def kernel(x, W_base, A, B):
  batch, seq = x.shape
  vocab, embed_dim = W_base.shape
  rank = A.shape[0]
  n_tokens = batch * seq

  x1d = x.reshape(n_tokens).astype(jnp.int32)
  w128 = _pack_rows_tc(W_base)
  a_t = A.T
  bt1d = B.T.reshape(rank * embed_dim)

  out = _sc_embed_lora(x1d, w128, a_t, bt1d, n_tokens, embed_dim, rank,
                       vocab // 2)
  return out.reshape(batch, seq, embed_dim)


# speedup vs baseline: 1.2977x; 1.0124x over previous
import functools

import jax
import jax.numpy as jnp
from jax import lax
from jax.experimental import pallas as pl
from jax.experimental.pallas import tpu as pltpu
from jax.experimental.pallas import tpu_sc as plsc

_NC = 2
_NS = 16
_NW = _NC * _NS

_TC = 256
_IDX_MINOR = 128


def _mesh():
  return plsc.VectorSubcoreMesh(
      core_axis_name="c", subcore_axis_name="s",
      num_cores=_NC, num_subcores=_NS)


def _sc_lora(x1d, a_t, bt1d, n_tokens, embed_dim, rank):
  tokens_per_worker = n_tokens // _NW
  chunks = tokens_per_worker // _TC
  groups = _TC // _IDX_MINOR
  jblocks = embed_dim // 16
  tgroups = _TC // 16

  @functools.partial(
      pl.kernel,
      out_type=jax.ShapeDtypeStruct((n_tokens, embed_dim), jnp.float32),
      mesh=_mesh(),
      compiler_params=pltpu.CompilerParams(use_tc_tiling_on_sc=False),
      scratch_types=[
          pltpu.VMEM((2, _TC), jnp.int32),
          pltpu.VMEM((2, _TC, rank), jnp.float32),
          pltpu.VMEM((2, _TC, embed_dim), jnp.float32),
          pltpu.VMEM((rank * embed_dim,), jnp.float32),
          pltpu.SemaphoreType.DMA((2,)),
          pltpu.SemaphoreType.DMA((2,)),
      ],
  )
  def k(x_hbm, at_hbm, bt_hbm, out_hbm, idx_v, a_v, out_v, bt_v, gsem, osem):
    wid = lax.axis_index("s") * _NC + lax.axis_index("c")
    pltpu.sync_copy(bt_hbm, bt_v)
    worker_tok0 = wid * tokens_per_worker

    def gather_parts(b):
      parts = []
      for g in range(groups):
        sl = pl.ds(g * _IDX_MINOR, _IDX_MINOR)
        parts.append((at_hbm.at[idx_v.at[b].at[sl]],
                      a_v.at[b].at[sl], gsem.at[b]))
      return parts

    def fire(c, b):
      tok0 = pl.multiple_of(worker_tok0 + c * _TC, _TC)
      pltpu.sync_copy(x_hbm.at[pl.ds(tok0, _TC)], idx_v.at[b])
      for src, dst, sem in gather_parts(b):
        pltpu.async_copy(src, dst, sem)

    btv = [[bt_v[pl.ds(r * embed_dim + j * 16, 16)] for j in range(jblocks)]
           for r in range(rank)]

    def compute(b):
      def grp1(g, carry):
        for kk in range(16):
          t = g * 16 + kk
          av = a_v[b, t, :]
          acc = [av[0] * btv[0][j] for j in range(jblocks)]
          for r in range(1, rank // 2):
            a = av[r]
            for j in range(jblocks):
              acc[j] = acc[j] + a * btv[r][j]
          for j in range(jblocks):
            out_v[b, t, pl.ds(j * 16, 16)] = acc[j]
        return carry

      lax.fori_loop(0, tgroups, grp1, 0, unroll=False)

      def grp2(g, carry):
        for kk in range(16):
          t = g * 16 + kk
          av = a_v[b, t, :]
          acc = [out_v[b, t, pl.ds(j * 16, 16)] for j in range(jblocks)]
          for r in range(rank // 2, rank):
            a = av[r]
            for j in range(jblocks):
              acc[j] = acc[j] + a * btv[r][j]
          for j in range(jblocks):
            out_v[b, t, pl.ds(j * 16, 16)] = acc[j]
        return carry

      lax.fori_loop(0, tgroups, grp2, 0, unroll=False)

    def out_slice(c):
      tok0 = pl.multiple_of(worker_tok0 + c * _TC, _TC)
      return out_hbm.at[pl.ds(tok0, _TC)]

    fire(0, 0)

    def chunk_body(c, carry):
      b = lax.rem(c, 2)
      nb = 1 - b
      nc = jnp.minimum(c + 1, chunks - 1)
      fire(nc, nb)
      for src, dst, sem in gather_parts(b):
        pltpu.make_async_copy(src, dst, sem).wait()

      @pl.when(c >= 2)
      def _():
        pltpu.make_async_copy(out_v.at[b], out_slice(c - 2), osem.at[b]).wait()

      compute(b)
      pltpu.async_copy(out_v.at[b], out_slice(c), osem.at[b])
      return carry

    lax.fori_loop(0, chunks, chunk_body, 0, unroll=False)

    for src, dst, sem in gather_parts(chunks % 2):
      pltpu.make_async_copy(src, dst, sem).wait()
    pltpu.make_async_copy(
        out_v.at[(chunks - 2) % 2], out_slice(chunks - 2),
        osem.at[(chunks - 2) % 2]).wait()
    pltpu.make_async_copy(
        out_v.at[(chunks - 1) % 2], out_slice(chunks - 1),
        osem.at[(chunks - 1) % 2]).wait()

  return k(x1d, a_t, bt1d)


def _sc_base_add(x1d, w128, lora, n_tokens, embed_dim, half_vocab):
  tokens_per_worker = n_tokens // _NW
  chunks = tokens_per_worker // _TC
  groups = _TC // _IDX_MINOR
  jblocks = embed_dim // 16
  tgroups = _TC // 16

  @functools.partial(
      pl.kernel,
      out_type=jax.ShapeDtypeStruct((n_tokens, embed_dim), jnp.float32),
      mesh=_mesh(),
      compiler_params=pltpu.CompilerParams(use_tc_tiling_on_sc=False),
      scratch_types=[
          pltpu.VMEM((2, _TC), jnp.int32),
          pltpu.VMEM((2, _TC), jnp.int32),
          pltpu.VMEM((2, _TC, 2 * embed_dim), jnp.float32),
          pltpu.VMEM((2, _TC, embed_dim), jnp.float32),
          pltpu.SemaphoreType.DMA((2,)),
          pltpu.SemaphoreType.DMA((2,)),
      ],
  )
  def k(x_hbm, w_hbm, l_hbm, out_hbm, idx_v, idxw_v, base_v, lor_v,
        gsem, osem):
    wid = lax.axis_index("s") * _NC + lax.axis_index("c")
    worker_tok0 = wid * tokens_per_worker

    def parts(c, b):
      tok0 = pl.multiple_of(worker_tok0 + c * _TC, _TC)
      ps = [(l_hbm.at[pl.ds(tok0, _TC)], lor_v.at[b], gsem.at[b])]
      for g in range(groups):
        sl = pl.ds(g * _IDX_MINOR, _IDX_MINOR)
        ps.append((w_hbm.at[idxw_v.at[b].at[sl]],
                   base_v.at[b].at[sl], gsem.at[b]))
      return ps

    def fire(c, b):
      tok0 = pl.multiple_of(worker_tok0 + c * _TC, _TC)
      pltpu.sync_copy(x_hbm.at[pl.ds(tok0, _TC)], idx_v.at[b])
      for i in range(_TC // 16):
        v = idx_v[b, pl.ds(i * 16, 16)]
        hi = 1 + ((v - half_vocab) >> 31)
        idxw_v[b, pl.ds(i * 16, 16)] = v - hi * half_vocab
      for src, dst, sem in parts(c, b):
        pltpu.async_copy(src, dst, sem)

    def compute(b):
      def grp(g, carry):
        vg = idx_v[b, pl.ds(g * 16, 16)]
        parg = (1 + ((vg - half_vocab) >> 31)) * embed_dim
        for kk in range(16):
          t = g * 16 + kk
          paroff = parg[kk]
          for j in range(jblocks):
            lor_v[b, t, pl.ds(j * 16, 16)] = (
                lor_v[b, t, pl.ds(j * 16, 16)]
                + base_v[b, t, pl.ds(paroff + j * 16, 16)])
        return carry

      lax.fori_loop(0, tgroups, grp, 0, unroll=False)

    def out_slice(c):
      tok0 = pl.multiple_of(worker_tok0 + c * _TC, _TC)
      return out_hbm.at[pl.ds(tok0, _TC)]

    fire(0, 0)

    def chunk_body(c, carry):
      b = lax.rem(c, 2)
      nb = 1 - b
      nc = jnp.minimum(c + 1, chunks - 1)
      fire(nc, nb)
      for src, dst, sem in parts(c, b):
        pltpu.make_async_copy(src, dst, sem).wait()

      @pl.when(c >= 2)
      def _():
        pltpu.make_async_copy(lor_v.at[b], out_slice(c - 2), osem.at[b]).wait()

      compute(b)
      pltpu.async_copy(lor_v.at[b], out_slice(c), osem.at[b])
      return carry

    lax.fori_loop(0, chunks, chunk_body, 0, unroll=False)

    for src, dst, sem in parts(chunks - 1, chunks % 2):
      pltpu.make_async_copy(src, dst, sem).wait()
    pltpu.make_async_copy(
        lor_v.at[(chunks - 2) % 2], out_slice(chunks - 2),
        osem.at[(chunks - 2) % 2]).wait()
    pltpu.make_async_copy(
        lor_v.at[(chunks - 1) % 2], out_slice(chunks - 1),
        osem.at[(chunks - 1) % 2]).wait()

  return k(x1d, w128, lora)


def kernel(x, W_base, A, B):
  batch, seq = x.shape
  vocab, embed_dim = W_base.shape
  rank = A.shape[0]
  n_tokens = batch * seq

  x1d = x.reshape(n_tokens).astype(jnp.int32)
  w128 = jnp.concatenate([W_base[:vocab // 2], W_base[vocab // 2:]], axis=1)
  a_t = A.T
  bt1d = B.T.reshape(rank * embed_dim)

  lora = _sc_lora(x1d, a_t, bt1d, n_tokens, embed_dim, rank)
  out = _sc_base_add(x1d, w128, lora, n_tokens, embed_dim, vocab // 2)
  return out.reshape(batch, seq, embed_dim)


# speedup vs baseline: 1.3583x; 1.0467x over previous
import functools

import jax
import jax.numpy as jnp
from jax import lax
from jax.experimental import pallas as pl
from jax.experimental.pallas import tpu as pltpu
from jax.experimental.pallas import tpu_sc as plsc

_NC = 2
_NS = 16
_NW = _NC * _NS

_TC = 256
_IDX_MINOR = 128


def _sc_embed_lora(x1d, w128, a_t, bt1d, n_tokens, embed_dim, rank):
  tokens_per_worker = n_tokens // _NW
  chunks = tokens_per_worker // _TC
  groups = _TC // _IDX_MINOR
  jblocks = embed_dim // 16
  tgroups = _TC // 16

  mesh = plsc.VectorSubcoreMesh(
      core_axis_name="c", subcore_axis_name="s",
      num_cores=_NC, num_subcores=_NS)

  @functools.partial(
      pl.kernel,
      out_type=jax.ShapeDtypeStruct((n_tokens, embed_dim), jnp.float32),
      mesh=mesh,
      compiler_params=pltpu.CompilerParams(use_tc_tiling_on_sc=False,
                                           needs_layout_passes=False),
      scratch_types=[
          pltpu.VMEM((2, _TC), jnp.int32),
          pltpu.VMEM((2, _TC), jnp.int32),
          pltpu.VMEM((2, _TC, 2 * embed_dim), jnp.float32),
          pltpu.VMEM((2, _TC, rank), jnp.float32),
          pltpu.VMEM((2, _TC, embed_dim), jnp.float32),
          pltpu.VMEM((rank * embed_dim,), jnp.bfloat16),
          pltpu.SemaphoreType.DMA((2,)),
          pltpu.SemaphoreType.DMA((2,)),
      ],
  )
  def k(x_hbm, w_hbm, at_hbm, bt_hbm, out_hbm,
        idx_v, idxw_v, base_v, a_v, out_v, bt_v, gsem, osem):
    wid = lax.axis_index("s") * _NC + lax.axis_index("c")
    pltpu.sync_copy(bt_hbm, bt_v)
    worker_tok0 = wid * tokens_per_worker

    def gather_parts(c, b):
      tok0 = pl.multiple_of(worker_tok0 + c * _TC, _TC)
      parts = []
      for g in range(groups):
        sl = pl.ds(g * _IDX_MINOR, _IDX_MINOR)
        parts.append((w_hbm.at[idxw_v.at[b].at[sl]],
                      base_v.at[b].at[sl], gsem.at[b]))
        parts.append((at_hbm.at[idx_v.at[b].at[sl]],
                      a_v.at[b].at[sl], gsem.at[b]))
      return tok0, parts

    def fire(c, b):
      tok0 = pl.multiple_of(worker_tok0 + c * _TC, _TC)
      pltpu.sync_copy(x_hbm.at[pl.ds(tok0, _TC)], idx_v.at[b])
      for i in range(_TC // 16):
        v = idx_v[b, pl.ds(i * 16, 16)]
        idxw_v[b, pl.ds(i * 16, 16)] = v >> 1
      _, parts = gather_parts(c, b)
      for src, dst, sem in parts:
        pltpu.async_copy(src, dst, sem)

    def wait_gathers(c, b):
      _, parts = gather_parts(c, b)
      for src, dst, sem in parts:
        pltpu.make_async_copy(src, dst, sem).wait()

    hblocks = embed_dim // 32
    btv = [[bt_v[pl.ds((r * hblocks + h) * 32, 32)] for h in range(hblocks)]
           for r in range(rank)]

    def compute(b):
      def grp(g, carry):
        parg = (idx_v[b, pl.ds(g * 16, 16)] & 1) * embed_dim
        for kk in range(16):
          t = g * 16 + kk
          paroff = parg[kk]
          av = a_v[b, t, :]

          def bf_splat(s):
            v16 = jnp.full((16,), s, dtype=jnp.float32)
            return plsc.pack(v16, v16, format=plsc.PackFormat.INTERLEAVED)

          acc = [bf_splat(av[0]) * btv[0][h] for h in range(hblocks)]
          for r in range(1, rank):
            a = bf_splat(av[r])
            for h in range(hblocks):
              acc[h] = acc[h] + a * btv[r][h]
          for h in range(hblocks):
            lo, hi = plsc.unpack(acc[h], format=plsc.PackFormat.INTERLEAVED)
            j0 = 2 * h
            out_v[b, t, pl.ds(j0 * 16, 16)] = (
                base_v[b, t, pl.ds(paroff + j0 * 16, 16)] + lo)
            out_v[b, t, pl.ds((j0 + 1) * 16, 16)] = (
                base_v[b, t, pl.ds(paroff + (j0 + 1) * 16, 16)] + hi)
        return carry

      lax.fori_loop(0, tgroups, grp, 0, unroll=False)

    def out_slice(c):
      tok0 = pl.multiple_of(worker_tok0 + c * _TC, _TC)
      return out_hbm.at[pl.ds(tok0, _TC)]

    fire(0, 0)

    def chunk_body(c, carry):
      b = lax.rem(c, 2)
      nb = 1 - b
      nc = jnp.minimum(c + 1, chunks - 1)
      fire(nc, nb)
      wait_gathers(c, b)

      @pl.when(c >= 2)
      def _():
        pltpu.make_async_copy(out_v.at[b], out_slice(c - 2), osem.at[b]).wait()

      compute(b)
      pltpu.async_copy(out_v.at[b], out_slice(c), osem.at[b])
      return carry

    lax.fori_loop(0, chunks, chunk_body, 0, unroll=False)

    wait_gathers(chunks - 1, chunks % 2)
    pltpu.make_async_copy(
        out_v.at[(chunks - 2) % 2], out_slice(chunks - 2),
        osem.at[(chunks - 2) % 2]).wait()
    pltpu.make_async_copy(
        out_v.at[(chunks - 1) % 2], out_slice(chunks - 1),
        osem.at[(chunks - 1) % 2]).wait()

  return k(x1d, w128, a_t, bt1d)


def kernel(x, W_base, A, B):
  batch, seq = x.shape
  vocab, embed_dim = W_base.shape
  rank = A.shape[0]
  n_tokens = batch * seq

  x1d = x.reshape(n_tokens).astype(jnp.int32)
  w128 = W_base.reshape(vocab // 2, 2 * embed_dim)
  a_t = A.T
  bt1d = (B.T.reshape(rank, embed_dim // 32, 2, 16)
          .transpose(0, 1, 3, 2)
          .reshape(rank * embed_dim)
          .astype(jnp.bfloat16))

  out = _sc_embed_lora(x1d, w128, a_t, bt1d, n_tokens, embed_dim, rank)
  return out.reshape(batch, seq, embed_dim)
